# Initial kernel scaffold; baseline (speedup 1.0000x reference)
import jax
import jax.numpy as jnp
from jax.experimental import pallas as pl


def kernel(idx, table):
    raise NotImplementedError("write your pallas kernel here")


# speedup vs baseline: 1.0342x; 1.0342x over previous
import functools

import jax
import jax.numpy as jnp
from jax import lax
from jax.experimental import pallas as pl
from jax.experimental.pallas import tpu as pltpu
from jax.experimental.pallas import tpu_sc as plsc


@functools.lru_cache(maxsize=None)
def _build_gather(B: int, V: int, D: int, chunk: int):
    info = plsc.get_sparse_core_info()
    nc, ns = info.num_cores, info.num_subcores
    nw = nc * ns
    assert B % nw == 0
    bpw = B // nw
    assert bpw % chunk == 0 and chunk % 8 == 0
    nchunk = bpw // chunk
    assert nchunk % 2 == 0 and nchunk >= 4

    mesh = plsc.VectorSubcoreMesh(core_axis_name="c", subcore_axis_name="s")

    @functools.partial(
        pl.kernel,
        mesh=mesh,
        compiler_params=pltpu.CompilerParams(use_tc_tiling_on_sc=False),
        out_type=jax.ShapeDtypeStruct((B, D), jnp.float32),
        scratch_types=[
            pltpu.VMEM((bpw,), jnp.int32),
            pltpu.VMEM((2, chunk, D), jnp.float32),
            pltpu.SemaphoreType.DMA,
            pltpu.SemaphoreType.DMA,
            pltpu.SemaphoreType.DMA,
            pltpu.SemaphoreType.DMA,
        ],
    )
    def k(idx_hbm, table_hbm, out_hbm, idx_v, rows_v, g0, g1, s0, s1):
        gsem = (g0, g1)
        ssem = (s0, s1)
        wid = lax.axis_index("s") * nc + lax.axis_index("c")
        base = pl.multiple_of(wid * bpw, bpw)
        pltpu.sync_copy(idx_hbm.at[pl.ds(base, bpw)], idx_v)

        def start_gather(i, b):
            off = pl.multiple_of(i * chunk, chunk)
            pltpu.async_copy(
                table_hbm.at[idx_v.at[pl.ds(off, chunk)]], rows_v.at[b], gsem[b]
            )

        def wait_gather(b):
            pltpu.make_async_copy(
                table_hbm.at[idx_v.at[pl.ds(0, chunk)]], rows_v.at[b], gsem[b]
            ).wait()

        def start_store(i, b):
            row0 = pl.multiple_of(base + i * chunk, chunk)
            pltpu.async_copy(
                rows_v.at[b], out_hbm.at[pl.ds(row0, chunk)], ssem[b]
            )

        def wait_store(b):
            pltpu.make_async_copy(
                rows_v.at[b], out_hbm.at[pl.ds(base, chunk)], ssem[b]
            ).wait()

        start_gather(0, 0)
        start_gather(1, 1)

        def body(j, carry):
            for b in range(2):
                i = j * 2 + b
                wait_gather(b)
                start_store(i, b)

                @pl.when(i + 2 < nchunk)
                def _():
                    wait_store(b)
                    start_gather(i + 2, b)

            return carry

        lax.fori_loop(0, nchunk // 2, body, 0)
        wait_store(0)
        wait_store(1)

    return k


def kernel(idx, table):
    batch, seq = idx.shape
    v, d = table.shape
    flat = idx.reshape(batch * seq)
    out = _build_gather(batch * seq, v, d, 40)(flat, table)
    return out.reshape(batch, seq, d)


# speedup vs baseline: 1.0364x; 1.0021x over previous
import functools

import jax
import jax.numpy as jnp
from jax import lax
from jax.experimental import pallas as pl
from jax.experimental.pallas import tpu as pltpu
from jax.experimental.pallas import tpu_sc as plsc


@functools.lru_cache(maxsize=None)
def _build_gather(BATCH: int, SEQ: int, V: int, D: int):
    info = plsc.get_sparse_core_info()
    nc, ns = info.num_cores, info.num_subcores
    nw = nc * ns
    assert BATCH % nw == 0
    bpw = BATCH // nw
    assert bpw % 2 == 0 and bpw >= 4

    mesh = plsc.VectorSubcoreMesh(core_axis_name="c", subcore_axis_name="s")

    @functools.partial(
        pl.kernel,
        mesh=mesh,
        compiler_params=pltpu.CompilerParams(use_tc_tiling_on_sc=False),
        out_type=jax.ShapeDtypeStruct((BATCH, SEQ, D), jnp.float32),
        scratch_types=[
            pltpu.VMEM((bpw, SEQ), jnp.int32),
            pltpu.VMEM((2, SEQ, D), jnp.float32),
            pltpu.SemaphoreType.DMA,
            pltpu.SemaphoreType.DMA,
            pltpu.SemaphoreType.DMA,
            pltpu.SemaphoreType.DMA,
        ],
    )
    def k(idx_hbm, table_hbm, out_hbm, idx_v, rows_v, g0, g1, s0, s1):
        gsem = (g0, g1)
        ssem = (s0, s1)
        wid = lax.axis_index("s") * nc + lax.axis_index("c")
        base = wid * bpw
        pltpu.sync_copy(idx_hbm.at[pl.ds(base, bpw)], idx_v)

        def start_gather(i, b):
            pltpu.async_copy(table_hbm.at[idx_v.at[i]], rows_v.at[b], gsem[b])

        def wait_gather(b):
            pltpu.make_async_copy(
                table_hbm.at[idx_v.at[0]], rows_v.at[b], gsem[b]
            ).wait()

        def start_store(i, b):
            pltpu.async_copy(rows_v.at[b], out_hbm.at[base + i], ssem[b])

        def wait_store(b):
            pltpu.make_async_copy(
                rows_v.at[b], out_hbm.at[base], ssem[b]
            ).wait()

        start_gather(0, 0)
        start_gather(1, 1)

        def body(j, carry):
            for b in range(2):
                i = j * 2 + b
                wait_gather(b)
                start_store(i, b)

                @pl.when(i + 2 < bpw)
                def _():
                    wait_store(b)
                    start_gather(i + 2, b)

            return carry

        lax.fori_loop(0, bpw // 2, body, 0)
        wait_store(0)
        wait_store(1)

    return k


def kernel(idx, table):
    batch, seq = idx.shape
    v, d = table.shape
    return _build_gather(batch, seq, v, d)(idx, table)
